# baseline (speedup 1.0000x reference)
import functools

import jax
import jax.numpy as jnp
from jax import lax
from jax.experimental import pallas as pl
from jax.experimental.pallas import tpu as pltpu
from jax.experimental.pallas import tpu_sc as plsc

B = 1
S = 2048
D = 1024
F = 1024
E = 8
V = 4096
N_TOK = B * S
TILE = 256
N_TILES = N_TOK // TILE
VS = 512
N_VS = V // VS


@functools.lru_cache(maxsize=None)
def _sc_gather_fn():
    info = plsc.get_sparse_core_info()
    nw = info.num_cores * info.num_subcores
    b_per_w = N_TOK // nw
    mesh = plsc.VectorSubcoreMesh(core_axis_name="c", subcore_axis_name="s")

    @functools.partial(
        pl.kernel,
        mesh=mesh,
        out_type=jax.ShapeDtypeStruct((N_TOK, D), jnp.float32),
        scratch_types=[
            pltpu.VMEM((b_per_w,), jnp.int32),
            pltpu.VMEM((b_per_w, D), jnp.float32),
            pltpu.SemaphoreType.DMA,
        ],
    )
    def gather_k(table_hbm, idx_hbm, out_hbm, idx_v, rows_v, sem):
        wid = lax.axis_index("s") * info.num_cores + lax.axis_index("c")
        base = wid * b_per_w
        pltpu.sync_copy(idx_hbm.at[pl.ds(base, b_per_w)], idx_v)
        pltpu.async_copy(table_hbm.at[idx_v], rows_v, sem).wait()
        pltpu.sync_copy(rows_v, out_hbm.at[pl.ds(base, b_per_w)])

    return gather_k


def _dot(a, b, dims):
    return lax.dot_general(a, b, (dims, ((), ())),
                           preferred_element_type=jnp.float32)


def _argmins(loads_ref):
    es = []
    for r in range(4):
        bv = loads_ref[r, 0]
        bi = jnp.int32(0)
        for c in range(1, E):
            v = loads_ref[r, c]
            pred = v < bv
            bi = jnp.where(pred, jnp.int32(c), bi)
            bv = jnp.where(pred, v, bv)
        es.append(bi)
    return es


def _fused_body(loads_ref, h0_ref, ids_ref, emb_ref,
                hw1, hw2, b1w1, b1w2, b2w1, b2w2, tw1, tw2,
                out_ref, wsc, hsc, stg, mrun, srun, crun, sems):
    p = pl.program_id(0)
    t = pl.program_id(1)
    w_hbms = (hw1, hw2, b1w1, b1w2, b2w1, b2w2, tw1, tw2)

    def _start(i, slot, es):
        pltpu.make_async_copy(w_hbms[i].at[es[i // 2]], stg.at[slot],
                              sems.at[slot]).start()

    def _land(j, slot):
        pltpu.make_async_copy(stg.at[slot], stg.at[slot],
                              sems.at[slot]).wait()
        wsc[j] = stg[slot].astype(jnp.bfloat16)

    @pl.when((p == 0) & (t == 0))
    def _stage0():
        es = _argmins(loads_ref)
        _start(0, 0, es)
        _start(1, 1, es)
        _land(0, 0)
        _start(2, 0, es)
        _land(1, 1)
        _start(3, 1, es)

    for ph in (1, 2):
        @pl.when((p == ph) & (t == 0))
        def _stage(ph=ph):
            es = _argmins(loads_ref)
            _land(2 * ph, 0)
            _start(2 * ph + 2, 0, es)
            _land(2 * ph + 1, 1)
            _start(2 * ph + 3, 1, es)

    @pl.when((p == 3) & (t == 0))
    def _stage3():
        _land(6, 0)
        _land(7, 1)

    @pl.when(p < 4)
    def _ffn():
        pc = jnp.minimum(p, 3)
        h_old = hsc[pl.ds(t * TILE, TILE), :]
        h_bf = jnp.where(p == 0, h0_ref[...].astype(jnp.bfloat16), h_old)
        a = jnp.maximum(_dot(h_bf, wsc[2 * pc], ((1,), (0,))), 0.0)
        h = _dot(a.astype(jnp.bfloat16), wsc[2 * pc + 1], ((1,), (0,)))
        hsc[pl.ds(t * TILE, TILE), :] = h.astype(jnp.bfloat16)

    @pl.when(p == 4)
    def _loss():
        eb = emb_ref[...].astype(jnp.bfloat16)
        lg = _dot(hsc[...], eb, ((1,), (1,)))
        lmax = jnp.max(lg, axis=1, keepdims=True)
        col = t * VS + lax.broadcasted_iota(jnp.int32, (N_TOK, VS), 1)
        csum = jnp.sum(jnp.where(col == ids_ref[...], lg, 0.0),
                       axis=1, keepdims=True)

        @pl.when(t == 0)
        def _init():
            mrun[...] = lmax
            srun[...] = jnp.sum(jnp.exp(lg - lmax), axis=1, keepdims=True)
            crun[...] = csum

        @pl.when(t != 0)
        def _update():
            m_old = mrun[...]
            m_new = jnp.maximum(m_old, lmax)
            srun[...] = (srun[...] * jnp.exp(m_old - m_new)
                         + jnp.sum(jnp.exp(lg - m_new), axis=1,
                                   keepdims=True))
            mrun[...] = m_new
            crun[...] += csum

        @pl.when(t == N_VS - 1)
        def _final():
            nll = mrun[...] + jnp.log(srun[...]) - crun[...]
            out_ref[...] = jnp.sum(nll, axis=0, keepdims=True) * (1.0 / N_TOK)


@functools.lru_cache(maxsize=None)
def _fused_fn():
    wspec = pl.BlockSpec(memory_space=pl.ANY)
    return pl.pallas_call(
        _fused_body,
        grid=(5, N_TILES),
        in_specs=[
            pl.BlockSpec(memory_space=pltpu.SMEM),
            pl.BlockSpec((TILE, D),
                         lambda p, t: (jnp.where(p == 0, t, 0), 0)),
            pl.BlockSpec((N_TOK, 1), lambda p, t: (0, 0)),
            pl.BlockSpec((VS, D),
                         lambda p, t: (jnp.where(p == 4, t, 0), 0)),
            wspec, wspec, wspec, wspec, wspec, wspec, wspec, wspec,
        ],
        out_specs=pl.BlockSpec((1, 1), lambda p, t: (0, 0)),
        out_shape=jax.ShapeDtypeStruct((1, 1), jnp.float32),
        scratch_shapes=[
            pltpu.VMEM((8, D, F), jnp.bfloat16),
            pltpu.VMEM((N_TOK, D), jnp.bfloat16),
            pltpu.VMEM((2, D, F), jnp.float32),
            pltpu.VMEM((N_TOK, 1), jnp.float32),
            pltpu.VMEM((N_TOK, 1), jnp.float32),
            pltpu.VMEM((N_TOK, 1), jnp.float32),
            pltpu.SemaphoreType.DMA((2,)),
        ],
        compiler_params=pltpu.CompilerParams(
            dimension_semantics=("arbitrary", "arbitrary"),
        ),
    )


def kernel(input_ids, loads, emb, head_w1, head_w2, body1_w1, body1_w2,
           body2_w1, body2_w2, tail_w1, tail_w2):
    ids = input_ids.reshape(-1)
    hidden0 = _sc_gather_fn()(emb, ids)
    ids2 = ids.reshape(N_TOK, 1)
    out = _fused_fn()(loads, hidden0, ids2, emb, head_w1, head_w2, body1_w1,
                      body1_w2, body2_w1, body2_w2, tail_w1, tail_w2)
    return out[0, 0]
